# Initial kernel scaffold; baseline (speedup 1.0000x reference)
import jax
import jax.numpy as jnp
from jax.experimental import pallas as pl


def kernel(x):
    raise NotImplementedError("write your pallas kernel here")


# speedup vs baseline: 2.3087x; 2.3087x over previous
import functools

import jax
import jax.numpy as jnp
from jax import lax
from jax.experimental import pallas as pl
from jax.experimental.pallas import tpu as pltpu
from jax.experimental.pallas import tpu_sc as plsc

R = 128
C = 32768
L = 16
NV = C // L
NC = 2
NS = 16
NW = NC * NS
RPW = R // NW


def _sort_a(v):
    return plsc.sort_key_val(v, v, descending=False)[0]


def _sort_d(v):
    return plsc.sort_key_val(v, v, descending=True)[0]


def _row_topk(buf_ref):
    neg = jnp.full((L,), -jnp.inf, dtype=jnp.float32)
    pos = jnp.full((L,), jnp.inf, dtype=jnp.float32)

    def body(i, carry):
        m0, m1, m2, m3, n0, n1, n2, n3 = carry
        v = buf_ref[pl.ds(i * L, L)]
        x = v
        m0n = jnp.maximum(m0, x); x = jnp.minimum(m0, x)
        m1n = jnp.maximum(m1, x); x = jnp.minimum(m1, x)
        m2n = jnp.maximum(m2, x); x = jnp.minimum(m2, x)
        m3n = jnp.maximum(m3, x)
        x = v
        n0n = jnp.minimum(n0, x); x = jnp.maximum(n0, x)
        n1n = jnp.minimum(n1, x); x = jnp.maximum(n1, x)
        n2n = jnp.minimum(n2, x); x = jnp.maximum(n2, x)
        n3n = jnp.minimum(n3, x)
        return (m0n, m1n, m2n, m3n, n0n, n1n, n2n, n3n)

    init = (neg, neg, neg, neg, pos, pos, pos, pos)
    m0, m1, m2, m3, n0, n1, n2, n3 = lax.fori_loop(0, NV, body, init,
                                                   unroll=8)

    u = jnp.maximum(_sort_d(m0), _sort_a(m1))
    w = jnp.maximum(_sort_d(m2), _sort_a(m3))
    t = jnp.maximum(_sort_d(u), _sort_a(w))
    top = _sort_d(t)

    u2 = jnp.minimum(_sort_a(n0), _sort_d(n1))
    w2 = jnp.minimum(_sort_a(n2), _sort_d(n3))
    t2 = jnp.minimum(_sort_a(u2), _sort_d(w2))
    bot = _sort_a(t2)

    return top + bot


_mesh = plsc.VectorSubcoreMesh(core_axis_name="c", subcore_axis_name="s")


@functools.partial(
    pl.kernel,
    mesh=_mesh,
    compiler_params=pltpu.CompilerParams(needs_layout_passes=False),
    out_type=jax.ShapeDtypeStruct((R, L), jnp.float32),
    scratch_types=[
        pltpu.VMEM((C,), jnp.float32),
        pltpu.VMEM((C,), jnp.float32),
        pltpu.VMEM((RPW, L), jnp.float32),
        pltpu.SemaphoreType.DMA,
        pltpu.SemaphoreType.DMA,
    ],
)
def _topk_sc(x_hbm, out_hbm, buf0, buf1, res, sem0, sem1):
    cid = lax.axis_index("c")
    sid = lax.axis_index("s")
    wid = sid * NC + cid
    base = wid * RPW

    bufs = (buf0, buf1)
    sems = (sem0, sem1)

    copies = [pltpu.async_copy(x_hbm.at[base], bufs[0], sems[0])]
    for r in range(RPW):
        if r + 1 < RPW:
            copies.append(
                pltpu.async_copy(x_hbm.at[base + r + 1],
                                 bufs[(r + 1) % 2], sems[(r + 1) % 2]))
        copies[r].wait()
        res[r, :] = _row_topk(bufs[r % 2])

    pltpu.sync_copy(res, out_hbm.at[pl.ds(base, RPW)])


@jax.jit
def kernel(x):
    out16 = _topk_sc(x)
    return out16[:, :4]
